# Initial kernel scaffold; baseline (speedup 1.0000x reference)
import jax
import jax.numpy as jnp
from jax.experimental import pallas as pl


def kernel(inputs, token_table, position_table):
    raise NotImplementedError("write your pallas kernel here")


# speedup vs baseline: 3.1167x; 3.1167x over previous
import functools

import jax
import jax.numpy as jnp
from jax import lax
from jax.experimental import pallas as pl
from jax.experimental.pallas import tpu as pltpu
from jax.experimental.pallas import tpu_sc as plsc

BATCH = 4096
SEQ = 200
DIM = 64

_info = plsc.get_sparse_core_info()
NC, NS, NL = _info.num_cores, _info.num_subcores, _info.num_lanes
NW = NC * NS
ROWS_PER_W = BATCH // NW


def _sc_embed(idx_hbm, tok_hbm, pos_hbm, out_hbm, idx_v, rows_v, pos_v, sem):
    wid = lax.axis_index("s") * NC + lax.axis_index("c")
    pltpu.sync_copy(pos_hbm, pos_v)

    def row_body(r, _):
        base = (wid * ROWS_PER_W + r) * SEQ
        pltpu.sync_copy(idx_hbm.at[pl.ds(base, SEQ)], idx_v)
        pltpu.async_copy(tok_hbm.at[idx_v], rows_v, sem).wait()

        def add_body(s, _):
            for c in range(DIM // NL):
                sl = pl.ds(c * NL, NL)
                rows_v[s, sl] = rows_v[s, sl] + pos_v[s, sl]
            return 0

        lax.fori_loop(0, SEQ, add_body, 0)
        pltpu.sync_copy(rows_v, out_hbm.at[pl.ds(base, SEQ)])
        return 0

    lax.fori_loop(0, ROWS_PER_W, row_body, 0)


@jax.jit
def kernel(inputs, token_table, position_table):
    idx_flat = inputs.reshape(-1).astype(jnp.int32)
    mesh = plsc.VectorSubcoreMesh(core_axis_name="c", subcore_axis_name="s")
    out = pl.kernel(
        _sc_embed,
        mesh=mesh,
        out_type=jax.ShapeDtypeStruct((BATCH * SEQ, DIM), jnp.float32),
        scratch_types=[
            pltpu.VMEM((SEQ,), jnp.int32),
            pltpu.VMEM((SEQ, DIM), jnp.float32),
            pltpu.VMEM((SEQ, DIM), jnp.float32),
            pltpu.SemaphoreType.DMA,
        ],
        compiler_params=pltpu.CompilerParams(use_tc_tiling_on_sc=False),
    )(idx_flat, token_table, position_table)
    return out.reshape(BATCH, SEQ, DIM)


# speedup vs baseline: 3.9652x; 1.2723x over previous
import functools

import jax
import jax.numpy as jnp
from jax import lax
from jax.experimental import pallas as pl
from jax.experimental.pallas import tpu as pltpu
from jax.experimental.pallas import tpu_sc as plsc

BATCH = 4096
SEQ = 200
DIM = 64

_info = plsc.get_sparse_core_info()
NC, NS, NL = _info.num_cores, _info.num_subcores, _info.num_lanes
NW = NC * NS
ROWS_PER_W = BATCH // NW
CH = 2
CHUNK = CH * SEQ
NBUF = 4
G = ROWS_PER_W // CH


def _sc_embed(idx_hbm, tok_hbm, pos_hbm, out_hbm, idx_bufs, row_bufs, in_sems,
              out_sems, pos_v):
    wid = lax.axis_index("s") * NC + lax.axis_index("c")
    wbase = wid * (ROWS_PER_W * SEQ)
    pltpu.sync_copy(pos_hbm, pos_v)

    def start_gather(g, b):
        base = wbase + g * CHUNK
        pltpu.sync_copy(idx_hbm.at[pl.ds(base, CHUNK)], idx_bufs[b])
        pltpu.async_copy(tok_hbm.at[idx_bufs[b]], row_bufs[b], in_sems[b])

    for b in range(NBUF - 1):
        start_gather(b, b)

    def outer(k, _):
        for b in range(NBUF):
            g = k * NBUF + b
            pltpu.make_async_copy(tok_hbm.at[idx_bufs[b]], row_bufs[b],
                                  in_sems[b]).wait()

            bp = (b + NBUF - 1) % NBUF

            @pl.when(jnp.logical_and(g >= 1, g + NBUF - 1 < G))
            def _():
                pltpu.make_async_copy(row_bufs[bp],
                                      out_hbm.at[pl.ds(0, CHUNK)],
                                      out_sems[bp]).wait()

            @pl.when(g + NBUF - 1 < G)
            def _():
                start_gather(g + NBUF - 1, bp)

            buf = row_bufs[b]

            @plsc.parallel_loop(0, SEQ, unroll=4)
            def _(s):
                for c in range(DIM // NL):
                    sl = pl.ds(c * NL, NL)
                    p = pos_v[s, sl]
                    for r in range(CH):
                        buf[r * SEQ + s, sl] = buf[r * SEQ + s, sl] + p

            pltpu.async_copy(buf, out_hbm.at[pl.ds(wbase + g * CHUNK, CHUNK)],
                             out_sems[b])
        return 0

    lax.fori_loop(0, G // NBUF, outer, 0)

    for b in range(NBUF):
        pltpu.make_async_copy(row_bufs[b], out_hbm.at[pl.ds(0, CHUNK)],
                              out_sems[b]).wait()


@jax.jit
def kernel(inputs, token_table, position_table):
    idx_flat = inputs.reshape(-1).astype(jnp.int32)
    mesh = plsc.VectorSubcoreMesh(core_axis_name="c", subcore_axis_name="s")
    out = pl.kernel(
        _sc_embed,
        mesh=mesh,
        out_type=jax.ShapeDtypeStruct((BATCH * SEQ, DIM), jnp.float32),
        scratch_types=[
            [pltpu.VMEM((CHUNK,), jnp.int32) for _ in range(NBUF)],
            [pltpu.VMEM((CHUNK, DIM), jnp.float32) for _ in range(NBUF)],
            [pltpu.SemaphoreType.DMA for _ in range(NBUF)],
            [pltpu.SemaphoreType.DMA for _ in range(NBUF)],
            pltpu.VMEM((SEQ, DIM), jnp.float32),
        ],
        compiler_params=pltpu.CompilerParams(use_tc_tiling_on_sc=False),
    )(idx_flat, token_table, position_table)
    return out.reshape(BATCH, SEQ, DIM)
